# Initial kernel scaffold; baseline (speedup 1.0000x reference)
import jax
import jax.numpy as jnp
from jax.experimental import pallas as pl


def kernel(input_mixed, ref_panel):
    raise NotImplementedError("write your pallas kernel here")


# speedup vs baseline: 25.6433x; 25.6433x over previous
import functools

import jax
import jax.numpy as jnp
from jax import lax
from jax.experimental import pallas as pl
from jax.experimental.pallas import tpu as pltpu
from jax.experimental.pallas import tpu_sc as plsc

B, C, N, L = 4, 3, 512, 4096
S = B * C
NW = 32
LCHUNK = L // NW
NGROUPS = LCHUNK // 16
UNROLL = 8

NEG_INF = float("-inf")


def _tec_body(mixed_hbm, ref_hbm, outmax_hbm, outidx_hbm,
              buf, m_all, omax, oidx):
    cid = lax.axis_index("c")
    sid = lax.axis_index("s")
    wid = sid * 2 + cid
    l0 = wid * LCHUNK

    pltpu.sync_copy(mixed_hbm.at[:, pl.ds(l0, LCHUNK)], m_all)

    def task(s, carry):
        b = s // C
        pltpu.sync_copy(ref_hbm.at[s, :, pl.ds(l0, LCHUNK)], buf)
        for g in range(NGROUPS):
            mg = m_all[b, pl.ds(g * 16, 16)]
            init = (jnp.full((16,), NEG_INF, jnp.float32),
                    jnp.full((16,), NEG_INF, jnp.float32),
                    jnp.zeros((16,), jnp.int32))

            def nblock(i, acc, g=g, mg=mg):
                mx1, mx2, idx = acc
                nb = i * UNROLL
                for u in range(UNROLL):
                    n = nb + u
                    r = buf[n, pl.ds(g * 16, 16)]
                    p = mg * r
                    gt = p > mx1
                    nv = jnp.full((16,), n, jnp.int32)
                    mx2 = jnp.where(gt, mx1, jnp.maximum(mx2, p))
                    mx1 = jnp.where(gt, p, mx1)
                    idx = jnp.where(gt, nv, idx)
                return mx1, mx2, idx

            mx1, mx2, idx = lax.fori_loop(0, N // UNROLL, nblock, init)
            omax[0, pl.ds(g * 16, 16)] = mx1
            omax[1, pl.ds(g * 16, 16)] = mx2
            oidx[pl.ds(g * 16, 16)] = idx
        pltpu.sync_copy(omax, outmax_hbm.at[s, :, pl.ds(l0, LCHUNK)])
        pltpu.sync_copy(oidx, outidx_hbm.at[s, pl.ds(l0, LCHUNK)])
        return carry

    lax.fori_loop(0, S, task, 0)


@jax.jit
def kernel(input_mixed, ref_panel):
    ref3 = ref_panel.reshape(S, N, L)
    run = pl.kernel(
        _tec_body,
        out_type=(jax.ShapeDtypeStruct((S, 2, L), jnp.float32),
                  jax.ShapeDtypeStruct((S, L), jnp.int32)),
        mesh=plsc.VectorSubcoreMesh(core_axis_name="c", subcore_axis_name="s"),
        scratch_types=[
            pltpu.VMEM((N, LCHUNK), jnp.float32),
            pltpu.VMEM((B, LCHUNK), jnp.float32),
            pltpu.VMEM((2, LCHUNK), jnp.float32),
            pltpu.VMEM((LCHUNK,), jnp.int32),
        ],
    )
    mx, idx = run(input_mixed, ref3)
    return mx.reshape(B, C, 2, L), idx.reshape(B, C, L)


# speedup vs baseline: 37.5893x; 1.4659x over previous
import functools

import jax
import jax.numpy as jnp
from jax import lax
from jax.experimental import pallas as pl
from jax.experimental.pallas import tpu as pltpu
from jax.experimental.pallas import tpu_sc as plsc

B, C, N, L = 4, 3, 512, 4096
S = B * C
NW = 32
LCHUNK = L // NW
NGROUPS = LCHUNK // 16
UNROLL = 8

NEG_INF = float("-inf")


NHALF = N // 2


def _tec_body(mixed_hbm, ref_hbm, outmax_hbm, outidx_hbm,
              bufs, m_all, acc_f, acc_i, omax_all, oidx_all,
              sem0, sem1):
    cid = lax.axis_index("c")
    sid = lax.axis_index("s")
    wid = sid * 2 + cid
    l0 = wid * LCHUNK
    sems = (sem0, sem1)

    def src(s, h):
        return ref_hbm.at[s, pl.ds(h * NHALF, NHALF), pl.ds(l0, LCHUNK)]

    pltpu.sync_copy(mixed_hbm.at[:, pl.ds(l0, LCHUNK)], m_all)

    pltpu.async_copy(src(0, 0), bufs.at[0], sems[0])

    def task(s, carry):
        b = s // C
        for h in (0, 1):
            pltpu.make_async_copy(src(s, h), bufs.at[h], sems[h]).wait()
            if h == 0:
                pltpu.async_copy(src(s, 1), bufs.at[1], sems[1])
            else:
                @pl.when(s < S - 1)
                def _():
                    pltpu.async_copy(src(s + 1, 0), bufs.at[0], sems[0])
            for g in range(NGROUPS):
                mg = m_all[b, pl.ds(g * 16, 16)]
                if h == 0:
                    init = (jnp.full((16,), NEG_INF, jnp.float32),
                            jnp.full((16,), NEG_INF, jnp.float32),
                            jnp.zeros((16,), jnp.int32))
                else:
                    init = (acc_f[0, g], acc_f[1, g], acc_i[g])

                def nblock(i, acc, h=h, g=g, mg=mg):
                    mx1, mx2, idx = acc
                    nb = h * NHALF + i * UNROLL
                    for u in range(UNROLL):
                        n = nb + u
                        r = bufs[h, i * UNROLL + u, pl.ds(g * 16, 16)]
                        p = mg * r
                        gt = p > mx1
                        nv = jnp.full((16,), n, jnp.int32)
                        mx2 = jnp.maximum(mx2, jnp.minimum(mx1, p))
                        mx1 = jnp.maximum(mx1, p)
                        idx = jnp.where(gt, nv, idx)
                    return mx1, mx2, idx

                mx1, mx2, idx = lax.fori_loop(0, NHALF // UNROLL, nblock, init)
                if h == 0:
                    acc_f[0, g] = mx1
                    acc_f[1, g] = mx2
                    acc_i[g] = idx
                else:
                    omax_all[s, 0, pl.ds(g * 16, 16)] = mx1
                    omax_all[s, 1, pl.ds(g * 16, 16)] = mx2
                    oidx_all[s, pl.ds(g * 16, 16)] = idx
        return carry

    lax.fori_loop(0, S, task, 0)

    pltpu.sync_copy(omax_all, outmax_hbm.at[:, :, pl.ds(l0, LCHUNK)])
    pltpu.sync_copy(oidx_all, outidx_hbm.at[:, pl.ds(l0, LCHUNK)])


@jax.jit
def kernel(input_mixed, ref_panel):
    ref3 = ref_panel.reshape(S, N, L)
    run = pl.kernel(
        _tec_body,
        out_type=(jax.ShapeDtypeStruct((S, 2, L), jnp.float32),
                  jax.ShapeDtypeStruct((S, L), jnp.int32)),
        mesh=plsc.VectorSubcoreMesh(core_axis_name="c", subcore_axis_name="s"),
        scratch_types=[
            pltpu.VMEM((2, NHALF, LCHUNK), jnp.float32),
            pltpu.VMEM((B, LCHUNK), jnp.float32),
            pltpu.VMEM((2, NGROUPS, 16), jnp.float32),
            pltpu.VMEM((NGROUPS, 16), jnp.int32),
            pltpu.VMEM((S, 2, LCHUNK), jnp.float32),
            pltpu.VMEM((S, LCHUNK), jnp.int32),
            pltpu.SemaphoreType.DMA,
            pltpu.SemaphoreType.DMA,
        ],
    )
    mx, idx = run(input_mixed, ref3)
    return mx.reshape(B, C, 2, L), idx.reshape(B, C, L)
